# baseline (speedup 1.0000x reference)
import functools

import jax
import jax.numpy as jnp
from jax import lax
from jax.experimental import pallas as pl
from jax.experimental.pallas import tpu as pltpu
from jax.experimental.pallas import tpu_sc as plsc

_NC = 2
_NS = 16
_NW = _NC * _NS
_CR = 32


def _sc_copy_body(kc, vc, ok, ov, buf0, buf1, ls0, ls1, ss0, ss1):
    wid = lax.axis_index("s") * _NC + lax.axis_index("c")
    rows_total = kc.shape[0]
    rows_w = rows_total // _NW
    base = wid * rows_w
    steps_per = rows_w // _CR
    plan = [(kc, ok, g) for g in range(steps_per)]
    plan += [(vc, ov, g) for g in range(steps_per)]
    bufs = (buf0, buf1)
    lsems = (ls0, ls1)
    ssems = (ss0, ss1)

    def load(t):
        src, _, g = plan[t]
        off = base + g * _CR
        return pltpu.make_async_copy(src.at[pl.ds(off, _CR)], bufs[t % 2], lsems[t % 2])

    def store(t):
        _, dst, g = plan[t]
        off = base + g * _CR
        return pltpu.make_async_copy(bufs[t % 2], dst.at[pl.ds(off, _CR)], ssems[t % 2])

    n = len(plan)
    load(0).start()
    for t in range(n):
        if t + 1 < n:
            if t - 1 >= 0:
                store(t - 1).wait()
            load(t + 1).start()
        load(t).wait()
        store(t).start()
    if n >= 2:
        store(n - 2).wait()
    store(n - 1).wait()


def _sc_copy(kc, vc):
    rows, F = kc.shape
    return pl.kernel(
        _sc_copy_body,
        out_type=[
            jax.ShapeDtypeStruct((rows, F), jnp.float32),
            jax.ShapeDtypeStruct((rows, F), jnp.float32),
        ],
        mesh=plsc.VectorSubcoreMesh(core_axis_name="c", subcore_axis_name="s"),
        scratch_types=[
            pltpu.VMEM((_CR, F), jnp.float32),
            pltpu.VMEM((_CR, F), jnp.float32),
            pltpu.SemaphoreType.DMA,
            pltpu.SemaphoreType.DMA,
            pltpu.SemaphoreType.DMA,
            pltpu.SemaphoreType.DMA,
        ],
    )(kc, vc)


def _tc_scatter_body(s_len, pos_ref, kv_ref, vv_ref, o1k_ref, o1v_ref,
                     ok_ref, ov_ref, sem):
    B, Q = kv_ref.shape[0], kv_ref.shape[1]
    for b in range(B):
        for i in range(Q):
            p = pos_ref[i]
            pltpu.make_async_copy(
                kv_ref.at[b, pl.ds(i, 1)], ok_ref.at[pl.ds(b * s_len + p, 1)], sem
            ).start()
            pltpu.make_async_copy(
                vv_ref.at[b, pl.ds(i, 1)], ov_ref.at[pl.ds(b * s_len + p, 1)], sem
            ).start()
    for b in range(B):
        for i in range(Q):
            p = pos_ref[i]
            pltpu.make_async_copy(
                kv_ref.at[b, pl.ds(i, 1)], ok_ref.at[pl.ds(b * s_len + p, 1)], sem
            ).wait()
            pltpu.make_async_copy(
                vv_ref.at[b, pl.ds(i, 1)], ov_ref.at[pl.ds(b * s_len + p, 1)], sem
            ).wait()


def _tc_scatter(input_pos, kv, vv, o1k, o1v, s_len):
    rows, F = o1k.shape
    hbm = pl.BlockSpec(memory_space=pltpu.MemorySpace.HBM)
    return pl.pallas_call(
        functools.partial(_tc_scatter_body, s_len),
        in_specs=[pl.BlockSpec(memory_space=pltpu.SMEM), hbm, hbm, hbm, hbm],
        out_specs=[hbm, hbm],
        out_shape=[
            jax.ShapeDtypeStruct((rows, F), jnp.float32),
            jax.ShapeDtypeStruct((rows, F), jnp.float32),
        ],
        input_output_aliases={3: 0, 4: 1},
        scratch_shapes=[pltpu.SemaphoreType.DMA],
    )(input_pos, kv, vv, o1k, o1v)


def kernel(input_pos, k_val, v_val, k_cache, v_cache):
    B, S, H, D = k_cache.shape
    Q = k_val.shape[1]
    F = H * D
    kc = k_cache.reshape(B * S, F)
    vc = v_cache.reshape(B * S, F)
    kv = k_val.reshape(B, Q, F)
    vv = v_val.reshape(B, Q, F)
    c_k, c_v = _sc_copy(kc, vc)
    out_k, out_v = _tc_scatter(input_pos, kv, vv, c_k, c_v, S)
    return (out_k.reshape(B, S, H, D), out_v.reshape(B, S, H, D))


# speedup vs baseline: 1.0001x; 1.0001x over previous
import functools

import jax
import jax.numpy as jnp
from jax import lax
from jax.experimental import pallas as pl
from jax.experimental.pallas import tpu as pltpu
from jax.experimental.pallas import tpu_sc as plsc

_NC = 2
_NS = 16
_NW = _NC * _NS
_CR = 32


def _sc_copy_body(kc, vc, ok, ov, buf0, buf1, ls0, ls1, ss0, ss1):
    wid = lax.axis_index("s") * _NC + lax.axis_index("c")
    rows_total = kc.shape[0]
    rows_w = rows_total // _NW
    base = wid * rows_w
    steps_per = rows_w // _CR
    plan = [(kc, ok, g) for g in range(steps_per)]
    plan += [(vc, ov, g) for g in range(steps_per)]
    bufs = (buf0, buf1)
    lsems = (ls0, ls1)
    ssems = (ss0, ss1)

    def load(t):
        src, _, g = plan[t]
        off = base + g * _CR
        return pltpu.make_async_copy(src.at[pl.ds(off, _CR)], bufs[t % 2], lsems[t % 2])

    def store(t):
        _, dst, g = plan[t]
        off = base + g * _CR
        return pltpu.make_async_copy(bufs[t % 2], dst.at[pl.ds(off, _CR)], ssems[t % 2])

    n = len(plan)
    load(0).start()
    for t in range(n):
        if t + 1 < n:
            if t - 1 >= 0:
                store(t - 1).wait()
            load(t + 1).start()
        load(t).wait()
        store(t).start()
    if n >= 2:
        store(n - 2).wait()
    store(n - 1).wait()


def _sc_copy(kc, vc):
    rows, F = kc.shape
    return pl.kernel(
        _sc_copy_body,
        out_type=[
            jax.ShapeDtypeStruct((rows, F), jnp.float32),
            jax.ShapeDtypeStruct((rows, F), jnp.float32),
        ],
        mesh=plsc.VectorSubcoreMesh(core_axis_name="c", subcore_axis_name="s"),
        compiler_params=pltpu.CompilerParams(use_tc_tiling_on_sc=True),
        scratch_types=[
            pltpu.VMEM((_CR, F), jnp.float32),
            pltpu.VMEM((_CR, F), jnp.float32),
            pltpu.SemaphoreType.DMA,
            pltpu.SemaphoreType.DMA,
            pltpu.SemaphoreType.DMA,
            pltpu.SemaphoreType.DMA,
        ],
    )(kc, vc)


def _tc_scatter_body(s_len, pos_ref, kv_ref, vv_ref, o1k_ref, o1v_ref,
                     ok_ref, ov_ref, sem):
    B, Q = kv_ref.shape[0], kv_ref.shape[1]
    for b in range(B):
        for i in range(Q):
            p = pos_ref[i]
            pltpu.make_async_copy(
                kv_ref.at[b, pl.ds(i, 1)], ok_ref.at[pl.ds(b * s_len + p, 1)], sem
            ).start()
            pltpu.make_async_copy(
                vv_ref.at[b, pl.ds(i, 1)], ov_ref.at[pl.ds(b * s_len + p, 1)], sem
            ).start()
    for b in range(B):
        for i in range(Q):
            p = pos_ref[i]
            pltpu.make_async_copy(
                kv_ref.at[b, pl.ds(i, 1)], ok_ref.at[pl.ds(b * s_len + p, 1)], sem
            ).wait()
            pltpu.make_async_copy(
                vv_ref.at[b, pl.ds(i, 1)], ov_ref.at[pl.ds(b * s_len + p, 1)], sem
            ).wait()


def _tc_scatter(input_pos, kv, vv, o1k, o1v, s_len):
    rows, F = o1k.shape
    hbm = pl.BlockSpec(memory_space=pltpu.MemorySpace.HBM)
    return pl.pallas_call(
        functools.partial(_tc_scatter_body, s_len),
        in_specs=[pl.BlockSpec(memory_space=pltpu.SMEM), hbm, hbm, hbm, hbm],
        out_specs=[hbm, hbm],
        out_shape=[
            jax.ShapeDtypeStruct((rows, F), jnp.float32),
            jax.ShapeDtypeStruct((rows, F), jnp.float32),
        ],
        input_output_aliases={3: 0, 4: 1},
        scratch_shapes=[pltpu.SemaphoreType.DMA],
    )(input_pos, kv, vv, o1k, o1v)


def kernel(input_pos, k_val, v_val, k_cache, v_cache):
    B, S, H, D = k_cache.shape
    Q = k_val.shape[1]
    F = H * D
    kc = k_cache.reshape(B * S, F)
    vc = v_cache.reshape(B * S, F)
    kv = k_val.reshape(B, Q, F)
    vv = v_val.reshape(B, Q, F)
    c_k, c_v = _sc_copy(kc, vc)
    out_k, out_v = _tc_scatter(input_pos, kv, vv, c_k, c_v, S)
    return (out_k.reshape(B, S, H, D), out_v.reshape(B, S, H, D))
